# baseline (speedup 1.0000x reference)
import functools

import jax
import jax.numpy as jnp
from jax import lax
from jax.experimental import pallas as pl
from jax.experimental.pallas import tpu as pltpu
from jax.experimental.pallas import tpu_sc as plsc

D = 128
B = 16384
NC = 2
NS = 16
NW = NC * NS
BPW = B // NW
CH = 128
NCH = BPW // CH
LANES = 16
HALF = CH // 2


def _elu_plus1_rows(ref, p):
    @plsc.parallel_loop(0, CH, unroll=1)
    def _row(r):
        for j in range(D // LANES):
            c = j * LANES
            x = ref[p, r, c:c + LANES]
            ref[p, r, c:c + LANES] = jnp.where(x > 0.0, x + 1.0, jnp.exp(x))


def _make_kernel():
    mesh = plsc.VectorSubcoreMesh(core_axis_name="c", subcore_axis_name="s")

    @functools.partial(
        pl.kernel,
        mesh=mesh,
        out_type=jax.ShapeDtypeStruct((B, 2 * D), jnp.float32),
        scratch_types=[
            pltpu.VMEM((NCH, CH), jnp.int32),
            pltpu.VMEM((2, CH, D), jnp.float32),
            pltpu.VMEM((2, CH, D), jnp.float32),
            pltpu.SemaphoreType.DMA,
            pltpu.SemaphoreType.DMA,
            pltpu.SemaphoreType.DMA,
            pltpu.SemaphoreType.DMA,
        ],
    )
    def k(idx_hbm, mu_hbm, sg_hbm, out_hbm, idx_v, mu_b, sg_b,
          gs0, gs1, ws0, ws1):
        gsem = (gs0, gs1)
        wsem = (ws0, ws1)
        wid = lax.axis_index("s") * NC + lax.axis_index("c")
        base = wid * BPW

        pltpu.sync_copy(idx_hbm.at[wid], idx_v)

        gm = [None, None]
        gs_h = [None, None]
        w = [[], []]

        gm[0] = pltpu.async_copy(mu_hbm.at[idx_v.at[0]], mu_b.at[0], gsem[0])
        gs_h[0] = pltpu.async_copy(sg_hbm.at[idx_v.at[0]], sg_b.at[0], gsem[0])

        for c in range(NCH):
            p = c & 1
            q = p ^ 1
            if c + 1 < NCH:
                if c >= 1:
                    for h in w[q]:
                        h.wait()
                    w[q] = []
                gm[q] = pltpu.async_copy(
                    mu_hbm.at[idx_v.at[c + 1]], mu_b.at[q], gsem[q])
                gs_h[q] = pltpu.async_copy(
                    sg_hbm.at[idx_v.at[c + 1]], sg_b.at[q], gsem[q])
            row0 = base + c * CH
            gm[p].wait()
            w[p].append(pltpu.async_copy(
                mu_b.at[p], out_hbm.at[pl.ds(row0, CH), pl.ds(0, D)],
                wsem[p]))
            gs_h[p].wait()
            _elu_plus1_rows(sg_b, p)
            w[p].append(pltpu.async_copy(
                sg_b.at[p],
                out_hbm.at[pl.ds(row0, CH), pl.ds(D, D)], wsem[p]))

        for p in (0, 1):
            for h in w[p]:
                h.wait()

    return k


_sc_kernel = _make_kernel()


def kernel(idx, mu_weight, sigma_weight):
    idx3 = idx.astype(jnp.int32).reshape(NW, NCH, CH)
    return _sc_kernel(idx3, mu_weight, sigma_weight)


# speedup vs baseline: 1.0302x; 1.0302x over previous
import functools

import jax
import jax.numpy as jnp
from jax import lax
from jax.experimental import pallas as pl
from jax.experimental.pallas import tpu as pltpu
from jax.experimental.pallas import tpu_sc as plsc

D = 128
B = 16384
NC = 2
NS = 16
NW = NC * NS
BPW = B // NW
CH = 128
NCH = BPW // CH
LANES = 16
NBUF = 3


def _elu_plus1_rows(ref, p):
    @plsc.parallel_loop(0, CH, unroll=2)
    def _row(r):
        for j in range(D // LANES):
            c = j * LANES
            x = ref[p, r, c:c + LANES]
            ref[p, r, c:c + LANES] = jnp.where(x > 0.0, x + 1.0, jnp.exp(x))


def _make_kernel():
    mesh = plsc.VectorSubcoreMesh(core_axis_name="c", subcore_axis_name="s")

    @functools.partial(
        pl.kernel,
        mesh=mesh,
        out_type=jax.ShapeDtypeStruct((B, 2 * D), jnp.float32),
        scratch_types=[
            pltpu.VMEM((NCH, CH), jnp.int32),
            pltpu.VMEM((NBUF, CH, D), jnp.float32),
            pltpu.VMEM((NBUF, CH, D), jnp.float32),
            pltpu.SemaphoreType.DMA,
            pltpu.SemaphoreType.DMA,
            pltpu.SemaphoreType.DMA,
            pltpu.SemaphoreType.DMA,
            pltpu.SemaphoreType.DMA,
            pltpu.SemaphoreType.DMA,
        ],
    )
    def k(idx_hbm, mu_hbm, sg_hbm, out_hbm, idx_v, mu_b, sg_b,
          gs0, gs1, gs2, ws0, ws1, ws2):
        gsem = (gs0, gs1, gs2)
        wsem = (ws0, ws1, ws2)
        wid = lax.axis_index("s") * NC + lax.axis_index("c")
        base = wid * BPW

        pltpu.sync_copy(idx_hbm.at[wid], idx_v)

        def fire_gathers(c):
            p = c % NBUF
            hm = pltpu.async_copy(mu_hbm.at[idx_v.at[c]], mu_b.at[p], gsem[p])
            hs = pltpu.async_copy(sg_hbm.at[idx_v.at[c]], sg_b.at[p], gsem[p])
            return hm, hs

        g = [None] * NCH
        w = [None] * NCH

        g[0] = fire_gathers(0)
        g[1] = fire_gathers(1)

        for c in range(NCH):
            p = c % NBUF
            if c + 2 < NCH:
                if c >= 1:
                    w[c - 1][0].wait()
                    w[c - 1][1].wait()
                g[c + 2] = fire_gathers(c + 2)
            row0 = base + c * CH
            g[c][0].wait()
            wm = pltpu.async_copy(
                mu_b.at[p], out_hbm.at[pl.ds(row0, CH), pl.ds(0, D)],
                wsem[p])
            g[c][1].wait()
            _elu_plus1_rows(sg_b, p)
            ws = pltpu.async_copy(
                sg_b.at[p], out_hbm.at[pl.ds(row0, CH), pl.ds(D, D)],
                wsem[p])
            w[c] = (wm, ws)

        for c in range(max(0, NCH - NBUF), NCH):
            w[c][0].wait()
            w[c][1].wait()

    return k


_sc_kernel = _make_kernel()


def kernel(idx, mu_weight, sigma_weight):
    idx3 = idx.astype(jnp.int32).reshape(NW, NCH, CH)
    return _sc_kernel(idx3, mu_weight, sigma_weight)
